# baseline (speedup 1.0000x reference)
import functools

import jax
import jax.numpy as jnp
from jax.experimental import pallas as pl
from jax.experimental.pallas import tpu as pltpu

_CH = 4096
_NBUF = 8
_LOOKAHEAD = 7


def _mk(x_hbm, m_all, w_ref, b_ref, o_hbm, xbuf, obuf, insem, outsem, *, nchunks):
    w = w_ref[...]
    b2 = b_ref[...]

    for k in range(_LOOKAHEAD):
        pltpu.make_async_copy(
            x_hbm.at[pl.ds(k * _CH, _CH), :], xbuf.at[k], insem.at[k]
        ).start()

    def body(c, _):
        slot = jax.lax.rem(c, _NBUF)
        pltpu.make_async_copy(
            x_hbm.at[pl.ds(c * _CH, _CH), :], xbuf.at[slot], insem.at[slot]
        ).wait()

        @pl.when(c >= _NBUF)
        def _wait_out():
            cp = c - _NBUF
            pltpu.make_async_copy(
                obuf.at[slot], o_hbm.at[pl.ds(cp * _CH, _CH), :], outsem.at[slot]
            ).wait()

        mm = jnp.dot(xbuf[slot], w, preferred_element_type=jnp.float32)
        mcol = m_all[pl.ds(c, 1), :].reshape(_CH, 1)
        obuf[slot] = (mm + b2) * mcol

        pltpu.make_async_copy(
            obuf.at[slot], o_hbm.at[pl.ds(c * _CH, _CH), :], outsem.at[slot]
        ).start()

        @pl.when(c + _LOOKAHEAD < nchunks)
        def _start_in():
            cn = c + _LOOKAHEAD
            sn = jax.lax.rem(cn, _NBUF)
            pltpu.make_async_copy(
                x_hbm.at[pl.ds(cn * _CH, _CH), :], xbuf.at[sn], insem.at[sn]
            ).start()

        return 0

    jax.lax.fori_loop(0, nchunks, body, 0)

    for k in range(_NBUF):
        cc = nchunks - _NBUF + k
        pltpu.make_async_copy(
            obuf.at[cc % _NBUF],
            o_hbm.at[pl.ds(cc * _CH, _CH), :],
            outsem.at[cc % _NBUF],
        ).wait()


def kernel(x, amask, W, b):
    n, in_f = x.shape
    out_f = W.shape[0]
    nchunks = n // _CH
    mf = amask.astype(jnp.float32).reshape(nchunks, _CH)
    wt = W.T
    b2 = b.reshape(1, out_f)
    return pl.pallas_call(
        functools.partial(_mk, nchunks=nchunks),
        in_specs=[
            pl.BlockSpec(memory_space=pl.ANY),
            pl.BlockSpec((nchunks, _CH), lambda: (0, 0)),
            pl.BlockSpec((in_f, out_f), lambda: (0, 0)),
            pl.BlockSpec((1, out_f), lambda: (0, 0)),
        ],
        out_specs=pl.BlockSpec(memory_space=pl.ANY),
        out_shape=jax.ShapeDtypeStruct((n, out_f), jnp.float32),
        scratch_shapes=[
            pltpu.VMEM((_NBUF, _CH, in_f), jnp.float32),
            pltpu.VMEM((_NBUF, _CH, out_f), jnp.float32),
            pltpu.SemaphoreType.DMA((_NBUF,)),
            pltpu.SemaphoreType.DMA((_NBUF,)),
        ],
    )(x, mf, wt, b2)
